# Initial kernel scaffold; baseline (speedup 1.0000x reference)
import functools

import jax
import jax.numpy as jnp
from jax import lax
from jax.experimental import pallas as pl
from jax.experimental.pallas import tpu as pltpu
from jax.experimental.pallas import tpu_sc as plsc

B, N, D = 4, 2500, 128
R, E, L = 5, 16000, 2
NPAD = 2560
NC, NS = 2, 16
BPC = B // NC
ACC_ROWS = R * NPAD + 16
ROWS_PT = ACC_ROWS // NS
EPT = 5120
CH = EPT // 128
HW = 16
HACC = 2 * R * NPAD
H_PT = 1616
HROWS = NS * H_PT
HEPT = 10240
HCH = HEPT // 128

_f32 = jnp.float32
_i32 = jnp.int32

_MESH = plsc.VectorSubcoreMesh(
    core_axis_name="c", subcore_axis_name="s", num_cores=NC, num_subcores=NS
)


def _fill_zero(ref, nrows):
    @pl.loop(0, nrows)
    def _(i):
        for j in range(ref.shape[1] // 16):
            ref.at[pl.ds(i, 1), pl.ds(16 * j, 16)][...] = jnp.zeros((1, 16), _f32)


def _sc_g_body(x_hbm, gi_hbm, si_hbm, g_hbm, gidx_v, sidx_v, buf_v, zero_v, acc_sh):
    c = lax.axis_index("c")
    s = lax.axis_index("s")
    _fill_zero(zero_v, 128)
    for k in range(BPC):
        b = c * BPC + k
        base = s * ROWS_PT
        for j in range(6):
            pltpu.sync_copy(zero_v, acc_sh.at[pl.ds(base + 128 * j, 128)])
        pltpu.sync_copy(zero_v.at[pl.ds(0, 33)], acc_sh.at[pl.ds(base + 768, 33)])
        plsc.subcore_barrier()
        pltpu.sync_copy(gi_hbm.at[b, s], gidx_v)
        pltpu.sync_copy(si_hbm.at[b, s], sidx_v)

        @pl.loop(0, CH)
        def _(j):
            pltpu.sync_copy(x_hbm.at[gidx_v.at[j]], buf_v)
            pltpu.sync_copy(buf_v, acc_sh.at[sidx_v.at[j]], add=True)

        plsc.subcore_barrier()
        for j in range(6):
            pltpu.sync_copy(
                acc_sh.at[pl.ds(base + 128 * j, 128)],
                g_hbm.at[b, pl.ds(base + 128 * j, 128)],
            )
        pltpu.sync_copy(
            acc_sh.at[pl.ds(base + 768, 33)], g_hbm.at[b, pl.ds(base + 768, 33)]
        )
        plsc.subcore_barrier()


_sc_g = functools.partial(
    pl.kernel,
    _sc_g_body,
    out_type=jax.ShapeDtypeStruct((B, ACC_ROWS, D), _f32),
    mesh=_MESH,
    scratch_types=[
        pltpu.VMEM((CH, 128), _i32),
        pltpu.VMEM((CH, 128), _i32),
        pltpu.VMEM((128, D), _f32),
        pltpu.VMEM((128, D), _f32),
        pltpu.VMEM_SHARED((ACC_ROWS, D), _f32),
    ],
)()


def _sc_hist_body(hi_hbm, h_hbm, hidx_v, ones_v, zero_v, hacc_sh):
    c = lax.axis_index("c")
    s = lax.axis_index("s")
    _fill_zero(zero_v, 128)

    @pl.loop(0, 128)
    def _(i):
        ones_v.at[pl.ds(i, 1), pl.ds(0, 16)][...] = jnp.ones((1, 16), _f32)

    for k in range(BPC):
        b = c * BPC + k
        base = s * H_PT
        for j in range(12):
            pltpu.sync_copy(
                zero_v.at[:, pl.ds(0, HW)], hacc_sh.at[pl.ds(base + 128 * j, 128)]
            )
        pltpu.sync_copy(
            zero_v.at[pl.ds(0, 80), pl.ds(0, HW)],
            hacc_sh.at[pl.ds(base + 1536, 80)],
        )
        plsc.subcore_barrier()
        pltpu.sync_copy(hi_hbm.at[b, s], hidx_v)

        @pl.loop(0, HCH)
        def _(j):
            pltpu.sync_copy(ones_v, hacc_sh.at[hidx_v.at[j]], add=True)

        plsc.subcore_barrier()
        for j in range(12):
            pltpu.sync_copy(
                hacc_sh.at[pl.ds(base + 128 * j, 128)],
                h_hbm.at[b, pl.ds(base + 128 * j, 128)],
            )
        pltpu.sync_copy(
            hacc_sh.at[pl.ds(base + 1536, 80)], h_hbm.at[b, pl.ds(base + 1536, 80)]
        )
        plsc.subcore_barrier()


_sc_hist = functools.partial(
    pl.kernel,
    _sc_hist_body,
    out_type=jax.ShapeDtypeStruct((B, HROWS, HW), _f32),
    mesh=_MESH,
    scratch_types=[
        pltpu.VMEM((HCH, 128), _i32),
        pltpu.VMEM((128, HW), _f32),
        pltpu.VMEM((128, D), _f32),
        pltpu.VMEM_SHARED((HROWS, HW), _f32),
    ],
)()

BLK = 512
NB = NPAD // BLK


def _tc_body(x_ref, g0, g1, g2, g3, g4, h_ref, wr_ref, w0_ref, wrb_ref, w0b_ref,
             xo_ref, m_ref):
    g_refs = (g0, g1, g2, g3, g4)
    x_blk = x_ref[...]
    s = lax.dot_general(
        x_blk, w0_ref[...], (((1,), (1,)), ((), ())), preferred_element_type=_f32
    )
    s = s + w0b_ref[...]
    hist = h_ref[...]
    wr = wr_ref[...]
    wrb = wrb_ref[...]
    denom = jnp.ones((BLK,), _f32)
    mask = jnp.zeros((BLK,), _i32)
    for r in range(R):
        gr = g_refs[r][...][0]
        s = s + lax.dot_general(
            gr, wr[r], (((1,), (1,)), ((), ())), preferred_element_type=_f32
        )
        d_r = hist[0, r, :]
        c_r = hist[0, R + r, :]
        s = s + d_r[:, None] * wrb[r][None, :]
        denom = denom + d_r
        mask = mask + ((d_r + c_r) == 0.0).astype(_i32)
    xo_ref[...] = jnp.maximum(s / denom[:, None], 0.0)
    m_ref[...] = mask[:, None]


def _tc_combine(x, g, hist3, wr_l, w0_l, wrb_l, w0b_l):
    g_spec = lambda r: pl.BlockSpec((1, BLK, D), lambda b, i, r=r: (b, r * NB + i, 0))
    return pl.pallas_call(
        _tc_body,
        grid=(B, NB),
        in_specs=[
            pl.BlockSpec((BLK, D), lambda b, i: (b * NB + i, 0)),
            g_spec(0), g_spec(1), g_spec(2), g_spec(3), g_spec(4),
            pl.BlockSpec((1, 2 * R, BLK), lambda b, i: (b, 0, i)),
            pl.BlockSpec((R, D, D), lambda b, i: (0, 0, 0)),
            pl.BlockSpec((D, D), lambda b, i: (0, 0)),
            pl.BlockSpec((R, D), lambda b, i: (0, 0)),
            pl.BlockSpec((1, D), lambda b, i: (0, 0)),
        ],
        out_specs=[
            pl.BlockSpec((BLK, D), lambda b, i: (b * NB + i, 0)),
            pl.BlockSpec((BLK, 1), lambda b, i: (b * NB + i, 0)),
        ],
        out_shape=[
            jax.ShapeDtypeStruct((B * NPAD, D), _f32),
            jax.ShapeDtypeStruct((B * NPAD, 1), _i32),
        ],
    )(x, g, g, g, g, g, hist3, wr_l, w0_l, wrb_l, w0b_l)


def kernel(nodes, edge_index, section, W0_w, W0_b, Wr_w, Wr_b):
    del section
    x0 = jnp.pad(nodes, ((0, 0), (0, NPAD - N), (0, 0))).reshape(B * NPAD, D)
    rows = edge_index[:, :, 0, :].astype(_i32)
    cols = edge_index[:, :, 1, :].astype(_i32)
    roff = (jnp.arange(R, dtype=_i32) * NPAD)[None, :, None]
    boff = (jnp.arange(B, dtype=_i32) * NPAD)[:, None, None]
    si = (rows + roff).reshape(B, R * E)
    gi = (cols + boff).reshape(B, R * E)
    pad_e = NS * EPT - R * E
    si = jnp.concatenate(
        [si, jnp.full((B, pad_e), R * NPAD, _i32)], axis=1
    ).reshape(B, NS, CH, 128)
    gi = jnp.concatenate(
        [gi, jnp.broadcast_to(boff[:, 0], (B, pad_e))], axis=1
    ).reshape(B, NS, CH, 128)
    hi = jnp.concatenate(
        [
            (rows + roff).reshape(B, R * E),
            (R * NPAD + cols + roff).reshape(B, R * E),
            jnp.full((B, NS * HEPT - 2 * R * E), HACC, _i32),
        ],
        axis=1,
    ).reshape(B, NS, HCH, 128)

    hist_raw = _sc_hist(hi)
    hist3 = hist_raw[:, : 2 * R * NPAD, 0].reshape(B, 2 * R, NPAD)

    x = x0
    mask_flat = None
    for l in range(L):
        g = _sc_g(x, gi, si)
        x, mask_flat = _tc_combine(
            x, g, hist3, Wr_w[:, l], W0_w[l], Wr_b[:, l], W0_b[l][None, :]
        )
    xout = x.reshape(B, NPAD, D)[:, :N]
    masks = mask_flat.reshape(B, NPAD)[:, :N]
    return (xout, masks)


# speedup vs baseline: 6.5519x; 6.5519x over previous
import functools

import jax
import jax.numpy as jnp
from jax import lax
from jax.experimental import pallas as pl
from jax.experimental.pallas import tpu as pltpu
from jax.experimental.pallas import tpu_sc as plsc

B, N, D = 4, 2500, 128
R, E, L = 5, 16000, 2
NPAD = 2560
NC, NS = 2, 16
BPC = B // NC
GROWS = R * NPAD
ACC_ROWS = 3 * NPAD + 128
DUMP = 3 * NPAD
ZPT = ACC_ROWS // NS
CH1, CH2 = 24, 16
CH = CH1 + CH2
HACC = 2 * R * NPAD
H_PT = 1664
HROWS = NS * H_PT
HEPT = 10240
HCH = HEPT // 128

_f32 = jnp.float32
_i32 = jnp.int32

@functools.lru_cache(maxsize=None)
def _mesh():
    return plsc.VectorSubcoreMesh(
        core_axis_name="c", subcore_axis_name="s", num_cores=NC, num_subcores=NS
    )


def _fill_zero(ref, nrows):
    @pl.loop(0, nrows)
    def _(i):
        for j in range(ref.shape[1] // 16):
            ref.at[pl.ds(i, 1), pl.ds(16 * j, 16)][...] = jnp.zeros((1, 16), _f32)


def _sc_g_body(x_hbm, es_hbm, g_hbm, gidx_v, sidx_v, buf_v, zero_v, acc_sh):
    c = lax.axis_index("c")
    s = lax.axis_index("s")
    _fill_zero(zero_v, 128)
    def zero_acc(nrows_pt):
        base = pl.multiple_of(s * nrows_pt, 8)
        full, tail = nrows_pt // 128, nrows_pt % 128
        for j in range(full):
            pltpu.sync_copy(zero_v, acc_sh.at[pl.ds(base + 128 * j, 128)])
        if tail:
            pltpu.sync_copy(
                zero_v.at[pl.ds(0, tail)], acc_sh.at[pl.ds(base + 128 * full, tail)]
            )

    def writeout(b, nrows_pt, out_off):
        base = pl.multiple_of(s * nrows_pt, 8)
        full, tail = nrows_pt // 128, nrows_pt % 128
        for j in range(full):
            pltpu.sync_copy(
                acc_sh.at[pl.ds(base + 128 * j, 128)],
                g_hbm.at[b, pl.ds(out_off + base + 128 * j, 128)],
            )
        if tail:
            pltpu.sync_copy(
                acc_sh.at[pl.ds(base + 128 * full, tail)],
                g_hbm.at[b, pl.ds(out_off + base + 128 * full, tail)],
            )

    for k in range(BPC):
        b = c * BPC + k
        ebase = pl.multiple_of((b * NS + s) * CH, CH)
        pltpu.sync_copy(es_hbm.at[pl.ds(ebase, CH)], gidx_v)
        pltpu.sync_copy(es_hbm.at[pl.ds(B * NS * CH + ebase, CH)], sidx_v)
        zero_acc(ZPT)
        plsc.subcore_barrier()

        @pl.loop(0, CH1)
        def _(j):
            pltpu.sync_copy(x_hbm.at[gidx_v.at[j]], buf_v)
            pltpu.sync_copy(buf_v, acc_sh.at[sidx_v.at[j]], add=True)

        plsc.subcore_barrier()
        writeout(b, 3 * NPAD // NS, 0)
        plsc.subcore_barrier()
        zero_acc(2 * NPAD // NS)
        plsc.subcore_barrier()

        @pl.loop(CH1, CH)
        def _(j):
            pltpu.sync_copy(x_hbm.at[gidx_v.at[j]], buf_v)
            pltpu.sync_copy(buf_v, acc_sh.at[sidx_v.at[j]], add=True)

        plsc.subcore_barrier()
        writeout(b, 2 * NPAD // NS, 3 * NPAD)
        plsc.subcore_barrier()


@functools.lru_cache(maxsize=None)
def _sc_g():
    return pl.kernel(
        _sc_g_body,
        out_type=jax.ShapeDtypeStruct((B, GROWS, D), _f32),
        mesh=_mesh(),
        scratch_types=[
            pltpu.VMEM((CH, 128), _i32),
            pltpu.VMEM((CH, 128), _i32),
            pltpu.VMEM((128, D), _f32),
            pltpu.VMEM((128, D), _f32),
            pltpu.VMEM_SHARED((ACC_ROWS, D), _f32),
        ],
    )


def _sc_hist_body(hi_hbm, h_hbm, hidx_v, ones_v, zero_v, hacc_sh):
    c = lax.axis_index("c")
    s = lax.axis_index("s")

    @pl.loop(0, H_PT // 16)
    def _(i):
        zero_v.at[pl.ds(16 * i, 16)][...] = jnp.zeros((16,), _f32)

    @pl.loop(0, 8)
    def _(i):
        ones_v.at[pl.ds(16 * i, 16)][...] = jnp.ones((16,), _f32)

    for k in range(BPC):
        b = c * BPC + k
        base = pl.multiple_of(s * H_PT, H_PT)
        pltpu.sync_copy(zero_v, hacc_sh.at[pl.ds(base, H_PT)])
        plsc.subcore_barrier()
        hbase = pl.multiple_of((b * NS + s) * HCH, HCH)
        pltpu.sync_copy(hi_hbm.at[pl.ds(hbase, HCH)], hidx_v)

        @pl.loop(0, HCH)
        def _(j):
            pltpu.sync_copy(ones_v, hacc_sh.at[hidx_v.at[j]], add=True)

        plsc.subcore_barrier()
        hoff = pl.multiple_of(b * HROWS + base, 8)
        pltpu.sync_copy(hacc_sh.at[pl.ds(base, H_PT)], h_hbm.at[pl.ds(hoff, H_PT)])
        plsc.subcore_barrier()


@functools.lru_cache(maxsize=None)
def _sc_hist():
    return pl.kernel(
        _sc_hist_body,
        out_type=jax.ShapeDtypeStruct((B * HROWS,), _f32),
        mesh=_mesh(),
        scratch_types=[
            pltpu.VMEM((HCH, 128), _i32),
            pltpu.VMEM((128,), _f32),
            pltpu.VMEM((H_PT,), _f32),
            pltpu.VMEM_SHARED((HROWS,), _f32),
        ],
    )

BLK = 512
NB = NPAD // BLK


def _tc_body(x_ref, g0, g1, g2, g3, g4, h_ref, wr_ref, w0_ref, wrb_ref, w0b_ref,
             xo_ref, m_ref):
    g_refs = (g0, g1, g2, g3, g4)
    x_blk = x_ref[...]
    s = lax.dot_general(
        x_blk, w0_ref[...], (((1,), (1,)), ((), ())), preferred_element_type=_f32
    )
    s = s + w0b_ref[...]
    hist = h_ref[...]
    wr = wr_ref[...]
    wrb = wrb_ref[...]
    denom = jnp.ones((BLK,), _f32)
    mask = jnp.zeros((BLK,), _i32)
    for r in range(R):
        gr = g_refs[r][...][0]
        s = s + lax.dot_general(
            gr, wr[r], (((1,), (1,)), ((), ())), preferred_element_type=_f32
        )
        d_r = hist[0, r, :]
        c_r = hist[0, R + r, :]
        s = s + d_r[:, None] * wrb[r][None, :]
        denom = denom + d_r
        mask = mask + ((d_r + c_r) == 0.0).astype(_i32)
    xo_ref[...] = jnp.maximum(s / denom[:, None], 0.0)
    m_ref[...] = mask[:, None]


def _tc_combine(x, g, hist3, wr_l, w0_l, wrb_l, w0b_l):
    g_spec = lambda r: pl.BlockSpec((1, BLK, D), lambda b, i, r=r: (b, r * NB + i, 0))
    return pl.pallas_call(
        _tc_body,
        grid=(B, NB),
        in_specs=[
            pl.BlockSpec((BLK, D), lambda b, i: (b * NB + i, 0)),
            g_spec(0), g_spec(1), g_spec(2), g_spec(3), g_spec(4),
            pl.BlockSpec((1, 2 * R, BLK), lambda b, i: (b, 0, i)),
            pl.BlockSpec((R, D, D), lambda b, i: (0, 0, 0)),
            pl.BlockSpec((D, D), lambda b, i: (0, 0)),
            pl.BlockSpec((R, D), lambda b, i: (0, 0)),
            pl.BlockSpec((1, D), lambda b, i: (0, 0)),
        ],
        out_specs=[
            pl.BlockSpec((BLK, D), lambda b, i: (b * NB + i, 0)),
            pl.BlockSpec((BLK, 1), lambda b, i: (b * NB + i, 0)),
        ],
        out_shape=[
            jax.ShapeDtypeStruct((B * NPAD, D), _f32),
            jax.ShapeDtypeStruct((B * NPAD, 1), _i32),
        ],
    )(x, g, g, g, g, g, hist3, wr_l, w0_l, wrb_l, w0b_l)


def kernel(nodes, edge_index, section, W0_w, W0_b, Wr_w, Wr_b):
    del section
    x0 = jnp.pad(nodes, ((0, 0), (0, NPAD - N), (0, 0))).reshape(B * NPAD, D)
    rows = edge_index[:, :, 0, :].astype(_i32)
    cols = edge_index[:, :, 1, :].astype(_i32)
    roff = (jnp.arange(R, dtype=_i32) * NPAD)[None, :, None]
    boff = (jnp.arange(B, dtype=_i32) * NPAD)[:, None, None]

    def tiled(a, nch, padv):
        a = a.reshape(B, NS, -1)
        pad = jnp.broadcast_to(padv, (B, NS, nch * 128 - a.shape[2])).astype(_i32)
        return jnp.concatenate([a, pad], axis=2)

    si = jnp.concatenate(
        [
            tiled((rows[:, :3] + roff[:, :3]).reshape(B, 3 * E), CH1, DUMP),
            tiled((rows[:, 3:] + roff[:, :2]).reshape(B, 2 * E), CH2, DUMP),
        ],
        axis=2,
    ).reshape(B * NS * CH, 128)
    gi = jnp.concatenate(
        [
            tiled((cols[:, :3] + boff).reshape(B, 3 * E), CH1, boff),
            tiled((cols[:, 3:] + boff).reshape(B, 2 * E), CH2, boff),
        ],
        axis=2,
    ).reshape(B * NS * CH, 128)
    hi = jnp.concatenate(
        [
            (rows + roff).reshape(B, R * E),
            (R * NPAD + cols + roff).reshape(B, R * E),
            jnp.full((B, NS * HEPT - 2 * R * E), HACC, _i32),
        ],
        axis=1,
    ).reshape(B * NS * HCH, 128)

    hist_raw = _sc_hist()(hi).reshape(B, HROWS)
    hist3 = hist_raw[:, : 2 * R * NPAD].reshape(B, 2 * R, NPAD)

    x = x0
    mask_flat = None
    es = jnp.concatenate([gi, si], axis=0)
    for l in range(L):
        g = _sc_g()(x, es)
        x, mask_flat = _tc_combine(
            x, g, hist3, Wr_w[:, l], W0_w[l], Wr_b[:, l], W0_b[l][None, :]
        )
    xout = x.reshape(B, NPAD, D)[:, :N]
    masks = mask_flat.reshape(B, NPAD)[:, :N]
    return (xout, masks)


# speedup vs baseline: 7.5837x; 1.1575x over previous
import functools

import jax
import jax.numpy as jnp
from jax import lax
from jax.experimental import pallas as pl
from jax.experimental.pallas import tpu as pltpu
from jax.experimental.pallas import tpu_sc as plsc

B, N, D = 4, 2500, 128
R, E, L = 5, 16000, 2
NPAD = 2560
NC, NS = 2, 16
BPC = B // NC
GROWS = R * NPAD
ACC_ROWS = 3 * NPAD + 128
DUMP = 3 * NPAD
ZPT = ACC_ROWS // NS
CH1, CH2 = 24, 16
CH = CH1 + CH2
HACC = 2 * R * NPAD
H_PT = 1664
HROWS = NS * H_PT
HEPT = 10240
HCH = HEPT // 128

_f32 = jnp.float32
_i32 = jnp.int32

@functools.lru_cache(maxsize=None)
def _mesh():
    return plsc.VectorSubcoreMesh(
        core_axis_name="c", subcore_axis_name="s", num_cores=NC, num_subcores=NS
    )


def _fill_zero(ref, nrows):
    @pl.loop(0, nrows)
    def _(i):
        for j in range(ref.shape[1] // 16):
            ref.at[pl.ds(i, 1), pl.ds(16 * j, 16)][...] = jnp.zeros((1, 16), _f32)


def _sc_g_body(x_hbm, es_hbm, g_hbm, gidx_v, sidx_v, buf0_v, buf1_v, zero_v,
               acc_sh, sem0, sem1):
    c = lax.axis_index("c")
    s = lax.axis_index("s")
    _fill_zero(zero_v, 128)

    def gfire(j, buf, sem):
        pltpu.async_copy(x_hbm.at[gidx_v.at[j]], buf, sem)

    def gwait(j, buf, sem):
        pltpu.make_async_copy(x_hbm.at[gidx_v.at[j]], buf, sem).wait()

    def scat(j, buf):
        pltpu.sync_copy(buf, acc_sh.at[sidx_v.at[j]], add=True)

    def run_chunks(lo, n):
        gfire(lo, buf0_v, sem0)

        @pl.loop(0, n // 2 - 1)
        def _(i):
            j = lo + 2 * i
            gfire(j + 1, buf1_v, sem1)
            gwait(j, buf0_v, sem0)
            scat(j, buf0_v)
            gfire(j + 2, buf0_v, sem0)
            gwait(j + 1, buf1_v, sem1)
            scat(j + 1, buf1_v)

        j = lo + n - 2
        gfire(j + 1, buf1_v, sem1)
        gwait(j, buf0_v, sem0)
        scat(j, buf0_v)
        gwait(j + 1, buf1_v, sem1)
        scat(j + 1, buf1_v)
    def zero_acc(nrows_pt):
        base = pl.multiple_of(s * nrows_pt, 8)
        full, tail = nrows_pt // 128, nrows_pt % 128
        for j in range(full):
            pltpu.sync_copy(zero_v, acc_sh.at[pl.ds(base + 128 * j, 128)])
        if tail:
            pltpu.sync_copy(
                zero_v.at[pl.ds(0, tail)], acc_sh.at[pl.ds(base + 128 * full, tail)]
            )

    def writeout(b, nrows_pt, out_off):
        base = pl.multiple_of(s * nrows_pt, 8)
        full, tail = nrows_pt // 128, nrows_pt % 128
        for j in range(full):
            pltpu.sync_copy(
                acc_sh.at[pl.ds(base + 128 * j, 128)],
                g_hbm.at[b, pl.ds(out_off + base + 128 * j, 128)],
            )
        if tail:
            pltpu.sync_copy(
                acc_sh.at[pl.ds(base + 128 * full, tail)],
                g_hbm.at[b, pl.ds(out_off + base + 128 * full, tail)],
            )

    for k in range(BPC):
        b = c * BPC + k
        ebase = pl.multiple_of((b * NS + s) * CH, CH)
        pltpu.sync_copy(es_hbm.at[pl.ds(ebase, CH)], gidx_v)
        pltpu.sync_copy(es_hbm.at[pl.ds(B * NS * CH + ebase, CH)], sidx_v)
        zero_acc(ZPT)
        plsc.subcore_barrier()

        run_chunks(0, CH1)
        plsc.subcore_barrier()
        writeout(b, 3 * NPAD // NS, 0)
        plsc.subcore_barrier()
        zero_acc(2 * NPAD // NS)
        plsc.subcore_barrier()

        run_chunks(CH1, CH2)
        plsc.subcore_barrier()
        writeout(b, 2 * NPAD // NS, 3 * NPAD)
        plsc.subcore_barrier()


@functools.lru_cache(maxsize=None)
def _sc_g():
    return pl.kernel(
        _sc_g_body,
        out_type=jax.ShapeDtypeStruct((B, GROWS, D), _f32),
        mesh=_mesh(),
        scratch_types=[
            pltpu.VMEM((CH, 128), _i32),
            pltpu.VMEM((CH, 128), _i32),
            pltpu.VMEM((128, D), _f32),
            pltpu.VMEM((128, D), _f32),
            pltpu.VMEM((128, D), _f32),
            pltpu.VMEM_SHARED((ACC_ROWS, D), _f32),
            pltpu.SemaphoreType.DMA,
            pltpu.SemaphoreType.DMA,
        ],
    )


def _sc_hist_body(hi_hbm, h_hbm, hidx_v, ones_v, zero_v, hacc_sh):
    c = lax.axis_index("c")
    s = lax.axis_index("s")

    @pl.loop(0, H_PT // 16)
    def _(i):
        zero_v.at[pl.ds(16 * i, 16)][...] = jnp.zeros((16,), _f32)

    @pl.loop(0, 8)
    def _(i):
        ones_v.at[pl.ds(16 * i, 16)][...] = jnp.ones((16,), _f32)

    for k in range(BPC):
        b = c * BPC + k
        base = pl.multiple_of(s * H_PT, H_PT)
        pltpu.sync_copy(zero_v, hacc_sh.at[pl.ds(base, H_PT)])
        plsc.subcore_barrier()
        hbase = pl.multiple_of((b * NS + s) * HCH, HCH)
        pltpu.sync_copy(hi_hbm.at[pl.ds(hbase, HCH)], hidx_v)

        @pl.loop(0, HCH)
        def _(j):
            pltpu.sync_copy(ones_v, hacc_sh.at[hidx_v.at[j]], add=True)

        plsc.subcore_barrier()
        hoff = pl.multiple_of(b * HROWS + base, 8)
        pltpu.sync_copy(hacc_sh.at[pl.ds(base, H_PT)], h_hbm.at[pl.ds(hoff, H_PT)])
        plsc.subcore_barrier()


@functools.lru_cache(maxsize=None)
def _sc_hist():
    return pl.kernel(
        _sc_hist_body,
        out_type=jax.ShapeDtypeStruct((B * HROWS,), _f32),
        mesh=_mesh(),
        scratch_types=[
            pltpu.VMEM((HCH, 128), _i32),
            pltpu.VMEM((128,), _f32),
            pltpu.VMEM((H_PT,), _f32),
            pltpu.VMEM_SHARED((HROWS,), _f32),
        ],
    )

BLK = 512
NB = NPAD // BLK


def _tc_body(x_ref, g0, g1, g2, g3, g4, h_ref, wr_ref, w0_ref, wrb_ref, w0b_ref,
             xo_ref, m_ref):
    g_refs = (g0, g1, g2, g3, g4)
    x_blk = x_ref[...]
    s = lax.dot_general(
        x_blk, w0_ref[...], (((1,), (1,)), ((), ())), preferred_element_type=_f32
    )
    s = s + w0b_ref[...]
    hist = h_ref[...]
    wr = wr_ref[...]
    wrb = wrb_ref[...]
    denom = jnp.ones((BLK,), _f32)
    mask = jnp.zeros((BLK,), _i32)
    for r in range(R):
        gr = g_refs[r][...][0]
        s = s + lax.dot_general(
            gr, wr[r], (((1,), (1,)), ((), ())), preferred_element_type=_f32
        )
        d_r = hist[0, r, :]
        c_r = hist[0, R + r, :]
        s = s + d_r[:, None] * wrb[r][None, :]
        denom = denom + d_r
        mask = mask + ((d_r + c_r) == 0.0).astype(_i32)
    xo_ref[...] = jnp.maximum(s / denom[:, None], 0.0)
    m_ref[...] = mask[:, None]


def _tc_combine(x, g, hist3, wr_l, w0_l, wrb_l, w0b_l):
    g_spec = lambda r: pl.BlockSpec((1, BLK, D), lambda b, i, r=r: (b, r * NB + i, 0))
    return pl.pallas_call(
        _tc_body,
        grid=(B, NB),
        in_specs=[
            pl.BlockSpec((BLK, D), lambda b, i: (b * NB + i, 0)),
            g_spec(0), g_spec(1), g_spec(2), g_spec(3), g_spec(4),
            pl.BlockSpec((1, 2 * R, BLK), lambda b, i: (b, 0, i)),
            pl.BlockSpec((R, D, D), lambda b, i: (0, 0, 0)),
            pl.BlockSpec((D, D), lambda b, i: (0, 0)),
            pl.BlockSpec((R, D), lambda b, i: (0, 0)),
            pl.BlockSpec((1, D), lambda b, i: (0, 0)),
        ],
        out_specs=[
            pl.BlockSpec((BLK, D), lambda b, i: (b * NB + i, 0)),
            pl.BlockSpec((BLK, 1), lambda b, i: (b * NB + i, 0)),
        ],
        out_shape=[
            jax.ShapeDtypeStruct((B * NPAD, D), _f32),
            jax.ShapeDtypeStruct((B * NPAD, 1), _i32),
        ],
    )(x, g, g, g, g, g, hist3, wr_l, w0_l, wrb_l, w0b_l)


def kernel(nodes, edge_index, section, W0_w, W0_b, Wr_w, Wr_b):
    del section
    x0 = jnp.pad(nodes, ((0, 0), (0, NPAD - N), (0, 0))).reshape(B * NPAD, D)
    rows = edge_index[:, :, 0, :].astype(_i32)
    cols = edge_index[:, :, 1, :].astype(_i32)
    roff = (jnp.arange(R, dtype=_i32) * NPAD)[None, :, None]
    boff = (jnp.arange(B, dtype=_i32) * NPAD)[:, None, None]

    def tiled(a, nch, padv):
        a = a.reshape(B, NS, -1)
        pad = jnp.broadcast_to(padv, (B, NS, nch * 128 - a.shape[2])).astype(_i32)
        return jnp.concatenate([a, pad], axis=2)

    si = jnp.concatenate(
        [
            tiled((rows[:, :3] + roff[:, :3]).reshape(B, 3 * E), CH1, DUMP),
            tiled((rows[:, 3:] + roff[:, :2]).reshape(B, 2 * E), CH2, DUMP),
        ],
        axis=2,
    ).reshape(B * NS * CH, 128)
    gi = jnp.concatenate(
        [
            tiled((cols[:, :3] + boff).reshape(B, 3 * E), CH1, boff),
            tiled((cols[:, 3:] + boff).reshape(B, 2 * E), CH2, boff),
        ],
        axis=2,
    ).reshape(B * NS * CH, 128)
    hi = jnp.concatenate(
        [
            (rows + roff).reshape(B, R * E),
            (R * NPAD + cols + roff).reshape(B, R * E),
            jnp.full((B, NS * HEPT - 2 * R * E), HACC, _i32),
        ],
        axis=1,
    ).reshape(B * NS * HCH, 128)

    hist_raw = _sc_hist()(hi).reshape(B, HROWS)
    hist3 = hist_raw[:, : 2 * R * NPAD].reshape(B, 2 * R, NPAD)

    x = x0
    mask_flat = None
    es = jnp.concatenate([gi, si], axis=0)
    for l in range(L):
        g = _sc_g()(x, es)
        x, mask_flat = _tc_combine(
            x, g, hist3, Wr_w[:, l], W0_w[l], Wr_b[:, l], W0_b[l][None, :]
        )
    xout = x.reshape(B, NPAD, D)[:, :N]
    masks = mask_flat.reshape(B, NPAD)[:, :N]
    return (xout, masks)


# speedup vs baseline: 7.5914x; 1.0010x over previous
import functools

import jax
import jax.numpy as jnp
from jax import lax
from jax.experimental import pallas as pl
from jax.experimental.pallas import tpu as pltpu
from jax.experimental.pallas import tpu_sc as plsc

B, N, D = 4, 2500, 128
R, E, L = 5, 16000, 2
NPAD = 2560
NC, NS = 2, 16
BPC = B // NC
GROWS = R * NPAD
ACC_ROWS = 3 * NPAD + 128
DUMP = 3 * NPAD
ZPT = ACC_ROWS // NS
CH1, CH2 = 24, 16
CH = CH1 + CH2
HACC = 2 * R * NPAD
H_PT = 1664
HROWS = NS * H_PT
HEPT = 10240
HCH = HEPT // 128

_f32 = jnp.float32
_i32 = jnp.int32

@functools.lru_cache(maxsize=None)
def _mesh():
    return plsc.VectorSubcoreMesh(
        core_axis_name="c", subcore_axis_name="s", num_cores=NC, num_subcores=NS
    )


def _fill_zero(ref, nrows):
    @pl.loop(0, nrows)
    def _(i):
        for j in range(ref.shape[1] // 16):
            ref.at[pl.ds(i, 1), pl.ds(16 * j, 16)][...] = jnp.zeros((1, 16), _f32)


def _sc_g_body(x_hbm, es_hbm, g_hbm, gidx_v, sidx_v, buf0_v, buf1_v, zero_v,
               acc_sh, sem0, sem1):
    c = lax.axis_index("c")
    s = lax.axis_index("s")
    _fill_zero(zero_v, 128)

    def gfire(j, buf, sem):
        pltpu.async_copy(x_hbm.at[gidx_v.at[j]], buf, sem)

    def gwait(j, buf, sem):
        pltpu.make_async_copy(x_hbm.at[gidx_v.at[j]], buf, sem).wait()

    def scat(j, buf):
        pltpu.sync_copy(buf, acc_sh.at[sidx_v.at[j]], add=True)

    def run_chunks(lo, n):
        gfire(lo, buf0_v, sem0)

        @pl.loop(0, n // 2 - 1)
        def _(i):
            j = lo + 2 * i
            gfire(j + 1, buf1_v, sem1)
            gwait(j, buf0_v, sem0)
            scat(j, buf0_v)
            gfire(j + 2, buf0_v, sem0)
            gwait(j + 1, buf1_v, sem1)
            scat(j + 1, buf1_v)

        j = lo + n - 2
        gfire(j + 1, buf1_v, sem1)
        gwait(j, buf0_v, sem0)
        scat(j, buf0_v)
        gwait(j + 1, buf1_v, sem1)
        scat(j + 1, buf1_v)
    def zero_acc(nrows_pt):
        base = pl.multiple_of(s * nrows_pt, 8)
        full, tail = nrows_pt // 128, nrows_pt % 128
        for j in range(full):
            pltpu.sync_copy(zero_v, acc_sh.at[pl.ds(base + 128 * j, 128)])
        if tail:
            pltpu.sync_copy(
                zero_v.at[pl.ds(0, tail)], acc_sh.at[pl.ds(base + 128 * full, tail)]
            )

    def writeout(b, nrows_pt, out_off):
        base = pl.multiple_of(s * nrows_pt, 8)
        full, tail = nrows_pt // 128, nrows_pt % 128
        for j in range(full):
            pltpu.sync_copy(
                acc_sh.at[pl.ds(base + 128 * j, 128)],
                g_hbm.at[b, pl.ds(out_off + base + 128 * j, 128)],
            )
        if tail:
            pltpu.sync_copy(
                acc_sh.at[pl.ds(base + 128 * full, tail)],
                g_hbm.at[b, pl.ds(out_off + base + 128 * full, tail)],
            )

    ebase = pl.multiple_of((c * NS + s) * CH, CH)
    pltpu.sync_copy(es_hbm.at[pl.ds(ebase, CH)], gidx_v)
    pltpu.sync_copy(es_hbm.at[pl.ds(2 * NS * CH + ebase, CH)], sidx_v)
    zero_acc(ZPT)
    plsc.subcore_barrier()

    run_chunks(0, CH1)
    plsc.subcore_barrier()
    writeout(c, 3 * NPAD // NS, 0)
    plsc.subcore_barrier()
    zero_acc(2 * NPAD // NS)
    plsc.subcore_barrier()

    run_chunks(CH1, CH2)
    plsc.subcore_barrier()
    writeout(c, 2 * NPAD // NS, 3 * NPAD)
    plsc.subcore_barrier()


@functools.lru_cache(maxsize=None)
def _sc_g():
    return pl.kernel(
        _sc_g_body,
        out_type=jax.ShapeDtypeStruct((NC, GROWS, D), _f32),
        mesh=_mesh(),
        scratch_types=[
            pltpu.VMEM((CH, 128), _i32),
            pltpu.VMEM((CH, 128), _i32),
            pltpu.VMEM((128, D), _f32),
            pltpu.VMEM((128, D), _f32),
            pltpu.VMEM((128, D), _f32),
            pltpu.VMEM_SHARED((ACC_ROWS, D), _f32),
            pltpu.SemaphoreType.DMA,
            pltpu.SemaphoreType.DMA,
        ],
    )


def _sc_hist_body(hi_hbm, h_hbm, hidx_v, ones_v, zero_v, hacc_sh):
    c = lax.axis_index("c")
    s = lax.axis_index("s")

    @pl.loop(0, H_PT // 16)
    def _(i):
        zero_v.at[pl.ds(16 * i, 16)][...] = jnp.zeros((16,), _f32)

    @pl.loop(0, 8)
    def _(i):
        ones_v.at[pl.ds(16 * i, 16)][...] = jnp.ones((16,), _f32)

    for k in range(BPC):
        b = c * BPC + k
        base = pl.multiple_of(s * H_PT, H_PT)
        pltpu.sync_copy(zero_v, hacc_sh.at[pl.ds(base, H_PT)])
        plsc.subcore_barrier()
        hbase = pl.multiple_of((b * NS + s) * HCH, HCH)
        pltpu.sync_copy(hi_hbm.at[pl.ds(hbase, HCH)], hidx_v)

        @pl.loop(0, HCH)
        def _(j):
            pltpu.sync_copy(ones_v, hacc_sh.at[hidx_v.at[j]], add=True)

        plsc.subcore_barrier()
        hoff = pl.multiple_of(b * HROWS + base, 8)
        pltpu.sync_copy(hacc_sh.at[pl.ds(base, H_PT)], h_hbm.at[pl.ds(hoff, H_PT)])
        plsc.subcore_barrier()


@functools.lru_cache(maxsize=None)
def _sc_hist():
    return pl.kernel(
        _sc_hist_body,
        out_type=jax.ShapeDtypeStruct((B * HROWS,), _f32),
        mesh=_mesh(),
        scratch_types=[
            pltpu.VMEM((HCH, 128), _i32),
            pltpu.VMEM((128,), _f32),
            pltpu.VMEM((H_PT,), _f32),
            pltpu.VMEM_SHARED((HROWS,), _f32),
        ],
    )

BLK = 512
NB = NPAD // BLK


def _tc_body(x_ref, g0, g1, g2, g3, g4, h_ref, wr_ref, w0_ref, wrb_ref, w0b_ref,
             xo_ref, m_ref):
    g_refs = (g0, g1, g2, g3, g4)
    x_blk = x_ref[...]
    s = lax.dot_general(
        x_blk, w0_ref[...], (((1,), (1,)), ((), ())), preferred_element_type=_f32
    )
    s = s + w0b_ref[...]
    hist = h_ref[...]
    wr = wr_ref[...]
    wrb = wrb_ref[...]
    denom = jnp.ones((BLK,), _f32)
    mask = jnp.zeros((BLK,), _i32)
    for r in range(R):
        gr = g_refs[r][...][0]
        s = s + lax.dot_general(
            gr, wr[r], (((1,), (1,)), ((), ())), preferred_element_type=_f32
        )
        d_r = hist[0, r, :]
        c_r = hist[0, R + r, :]
        s = s + d_r[:, None] * wrb[r][None, :]
        denom = denom + d_r
        mask = mask + ((d_r + c_r) == 0.0).astype(_i32)
    xo_ref[...] = jnp.maximum(s / denom[:, None], 0.0)
    m_ref[...] = mask[:, None]


def _tc_combine(x, g, hist3, wr_l, w0_l, wrb_l, w0b_l):
    g_spec = lambda r: pl.BlockSpec((1, BLK, D), lambda b, i, r=r: (b, r * NB + i, 0))
    return pl.pallas_call(
        _tc_body,
        grid=(NC, NB),
        in_specs=[
            pl.BlockSpec((BLK, D), lambda b, i: (b * NB + i, 0)),
            g_spec(0), g_spec(1), g_spec(2), g_spec(3), g_spec(4),
            pl.BlockSpec((1, 2 * R, BLK), lambda b, i: (b, 0, i)),
            pl.BlockSpec((R, D, D), lambda b, i: (0, 0, 0)),
            pl.BlockSpec((D, D), lambda b, i: (0, 0)),
            pl.BlockSpec((R, D), lambda b, i: (0, 0)),
            pl.BlockSpec((1, D), lambda b, i: (0, 0)),
        ],
        out_specs=[
            pl.BlockSpec((BLK, D), lambda b, i: (b * NB + i, 0)),
            pl.BlockSpec((BLK, 1), lambda b, i: (b * NB + i, 0)),
        ],
        out_shape=[
            jax.ShapeDtypeStruct((NC * NPAD, D), _f32),
            jax.ShapeDtypeStruct((NC * NPAD, 1), _i32),
        ],
    )(x, g, g, g, g, g, hist3, wr_l, w0_l, wrb_l, w0b_l)


def kernel(nodes, edge_index, section, W0_w, W0_b, Wr_w, Wr_b):
    del section
    xpad = jnp.pad(nodes, ((0, 0), (0, NPAD - N), (0, 0)))
    rows = edge_index[:, :, 0, :].astype(_i32)
    cols = edge_index[:, :, 1, :].astype(_i32)
    roff = (jnp.arange(R, dtype=_i32) * NPAD)[None, :, None]
    boff = ((jnp.arange(B, dtype=_i32) // 2) * NPAD)[:, None, None]

    def tiled(a, nch, padv):
        a = a.reshape(B, NS, -1)
        pad = jnp.broadcast_to(padv, (B, NS, nch * 128 - a.shape[2])).astype(_i32)
        return jnp.concatenate([a, pad], axis=2)

    si = jnp.concatenate(
        [
            tiled((rows[:, :3] + roff[:, :3]).reshape(B, 3 * E), CH1, DUMP),
            tiled((rows[:, 3:] + roff[:, :2]).reshape(B, 2 * E), CH2, DUMP),
        ],
        axis=2,
    ).reshape(B * NS * CH, 128)
    gi = jnp.concatenate(
        [
            tiled((cols[:, :3] + boff).reshape(B, 3 * E), CH1, boff),
            tiled((cols[:, 3:] + boff).reshape(B, 2 * E), CH2, boff),
        ],
        axis=2,
    ).reshape(B * NS * CH, 128)
    hi = jnp.concatenate(
        [
            (rows + roff).reshape(B, R * E),
            (R * NPAD + cols + roff).reshape(B, R * E),
            jnp.full((B, NS * HEPT - 2 * R * E), HACC, _i32),
        ],
        axis=1,
    ).reshape(B * NS * HCH, 128)

    hist_raw = _sc_hist()(hi).reshape(B, HROWS)
    hist4 = hist_raw[:, : 2 * R * NPAD].reshape(B, 2 * R, NPAD)

    gi4 = gi.reshape(B, NS * CH, 128)
    si4 = si.reshape(B, NS * CH, 128)
    es_p, x_p, h_p = [], [], []
    for p in range(2):
        sel = jnp.array([p, p + 2], dtype=_i32)
        es_p.append(
            jnp.concatenate(
                [gi4[sel].reshape(-1, 128), si4[sel].reshape(-1, 128)], axis=0
            )
        )
        x_p.append(xpad[sel].reshape(NC * NPAD, D))
        h_p.append(hist4[sel])

    mask_p = [None, None]
    for l in range(L):
        for p in range(2):
            g = _sc_g()(x_p[p], es_p[p])
            x_p[p], mask_p[p] = _tc_combine(
                x_p[p], g, h_p[p], Wr_w[:, l], W0_w[l], Wr_b[:, l], W0_b[l][None, :]
            )
    xq = jnp.stack([x_p[0], x_p[1]]).reshape(2, NC, NPAD, D)
    mq = jnp.stack([mask_p[0], mask_p[1]]).reshape(2, NC, NPAD)
    xout = xq.transpose(1, 0, 2, 3).reshape(B, NPAD, D)[:, :N]
    masks = mq.transpose(1, 0, 2).reshape(B, NPAD)[:, :N]
    return (xout, masks)


# speedup vs baseline: 7.7448x; 1.0202x over previous
import functools

import jax
import jax.numpy as jnp
from jax import lax
from jax.experimental import pallas as pl
from jax.experimental.pallas import tpu as pltpu
from jax.experimental.pallas import tpu_sc as plsc

B, N, D = 4, 2500, 128
R, E, L = 5, 16000, 2
NPAD = 2560
NC, NS = 2, 16
BPC = B // NC
GROWS = R * NPAD
ACC_ROWS = 3 * NPAD + 128
DUMP = 3 * NPAD
ZPT = ACC_ROWS // NS
CH1, CH2 = 24, 16
CH = CH1 + CH2
HACC = 2 * R * NPAD
H_PT = 1664
HROWS = NS * H_PT
HEPT = 10240
HCH = HEPT // 128

_f32 = jnp.float32
_i32 = jnp.int32

@functools.lru_cache(maxsize=None)
def _mesh():
    return plsc.VectorSubcoreMesh(
        core_axis_name="c", subcore_axis_name="s", num_cores=NC, num_subcores=NS
    )


def _fill_zero(ref, nrows):
    @pl.loop(0, nrows)
    def _(i):
        for j in range(ref.shape[1] // 16):
            ref.at[pl.ds(i, 1), pl.ds(16 * j, 16)][...] = jnp.zeros((1, 16), _f32)


def _sc_g_body(x_hbm, es_hbm, g_hbm, gidx_v, sidx_v, buf0_v, buf1_v, zero_v,
               acc_sh, sem0, sem1):
    c = lax.axis_index("c")
    s = lax.axis_index("s")
    _fill_zero(zero_v, 128)

    def gfire(j, buf, sem):
        pltpu.async_copy(x_hbm.at[gidx_v.at[j]], buf, sem)

    def gwait(j, buf, sem):
        pltpu.make_async_copy(x_hbm.at[gidx_v.at[j]], buf, sem).wait()

    def scat(j, buf):
        pltpu.sync_copy(buf, acc_sh.at[sidx_v.at[j]], add=True)

    def run_chunks(lo, n):
        gfire(lo, buf0_v, sem0)

        @pl.loop(0, n // 2 - 1)
        def _(i):
            j = lo + 2 * i
            gfire(j + 1, buf1_v, sem1)
            gwait(j, buf0_v, sem0)
            scat(j, buf0_v)
            gfire(j + 2, buf0_v, sem0)
            gwait(j + 1, buf1_v, sem1)
            scat(j + 1, buf1_v)

        j = lo + n - 2
        gfire(j + 1, buf1_v, sem1)
        gwait(j, buf0_v, sem0)
        scat(j, buf0_v)
        gwait(j + 1, buf1_v, sem1)
        scat(j + 1, buf1_v)
    def zero_acc(nrows_pt):
        base = pl.multiple_of(s * nrows_pt, 8)
        full, tail = nrows_pt // 128, nrows_pt % 128
        for j in range(full):
            pltpu.sync_copy(zero_v, acc_sh.at[pl.ds(base + 128 * j, 128)])
        if tail:
            pltpu.sync_copy(
                zero_v.at[pl.ds(0, tail)], acc_sh.at[pl.ds(base + 128 * full, tail)]
            )

    def writeout(b, nrows_pt, out_off):
        base = pl.multiple_of(s * nrows_pt, 8)
        full, tail = nrows_pt // 128, nrows_pt % 128
        for j in range(full):
            pltpu.sync_copy(
                acc_sh.at[pl.ds(base + 128 * j, 128)],
                g_hbm.at[b, pl.ds(out_off + base + 128 * j, 128)],
            )
        if tail:
            pltpu.sync_copy(
                acc_sh.at[pl.ds(base + 128 * full, tail)],
                g_hbm.at[b, pl.ds(out_off + base + 128 * full, tail)],
            )

    for k in range(BPC):
        b = c * BPC + k
        ebase = pl.multiple_of((b * NS + s) * CH, CH)
        pltpu.sync_copy(es_hbm.at[pl.ds(ebase, CH)], gidx_v)
        pltpu.sync_copy(es_hbm.at[pl.ds(B * NS * CH + ebase, CH)], sidx_v)
        zero_acc(ZPT)
        plsc.subcore_barrier()

        run_chunks(0, CH1)
        plsc.subcore_barrier()
        writeout(b, 3 * NPAD // NS, 0)
        plsc.subcore_barrier()
        zero_acc(2 * NPAD // NS)
        plsc.subcore_barrier()

        run_chunks(CH1, CH2)
        plsc.subcore_barrier()
        writeout(b, 2 * NPAD // NS, 3 * NPAD)
        plsc.subcore_barrier()


@functools.lru_cache(maxsize=None)
def _sc_g():
    return pl.kernel(
        _sc_g_body,
        out_type=jax.ShapeDtypeStruct((B, GROWS, D), _f32),
        mesh=_mesh(),
        scratch_types=[
            pltpu.VMEM((CH, 128), _i32),
            pltpu.VMEM((CH, 128), _i32),
            pltpu.VMEM((128, D), _f32),
            pltpu.VMEM((128, D), _f32),
            pltpu.VMEM((128, D), _f32),
            pltpu.VMEM_SHARED((ACC_ROWS, D), _f32),
            pltpu.SemaphoreType.DMA,
            pltpu.SemaphoreType.DMA,
        ],
    )


def _sc_hist_body(hi_hbm, h_hbm, hidx_v, ones_v, zero_v, hacc_sh):
    c = lax.axis_index("c")
    s = lax.axis_index("s")

    @pl.loop(0, H_PT // 16)
    def _(i):
        zero_v.at[pl.ds(16 * i, 16)][...] = jnp.zeros((16,), _f32)

    @pl.loop(0, 8)
    def _(i):
        ones_v.at[pl.ds(16 * i, 16)][...] = jnp.ones((16,), _f32)

    for k in range(BPC):
        b = c * BPC + k
        base = pl.multiple_of(s * H_PT, H_PT)
        pltpu.sync_copy(zero_v, hacc_sh.at[pl.ds(base, H_PT)])
        plsc.subcore_barrier()
        hbase = pl.multiple_of((b * NS + s) * HCH, HCH)
        pltpu.sync_copy(hi_hbm.at[pl.ds(hbase, HCH)], hidx_v)

        @pl.loop(0, HCH)
        def _(j):
            pltpu.sync_copy(ones_v, hacc_sh.at[hidx_v.at[j]], add=True)

        plsc.subcore_barrier()
        hoff = pl.multiple_of(b * HROWS + base, 8)
        pltpu.sync_copy(hacc_sh.at[pl.ds(base, H_PT)], h_hbm.at[pl.ds(hoff, H_PT)])
        plsc.subcore_barrier()


@functools.lru_cache(maxsize=None)
def _sc_hist():
    return pl.kernel(
        _sc_hist_body,
        out_type=jax.ShapeDtypeStruct((B * HROWS,), _f32),
        mesh=_mesh(),
        scratch_types=[
            pltpu.VMEM((HCH, 128), _i32),
            pltpu.VMEM((128,), _f32),
            pltpu.VMEM((H_PT,), _f32),
            pltpu.VMEM_SHARED((HROWS,), _f32),
        ],
    )

BLK = 512
NB = NPAD // BLK


def _tc_body(x_ref, g0, g1, g2, g3, g4, h_ref, wr_ref, w0_ref, wrb_ref, w0b_ref,
             xo_ref, m_ref):
    g_refs = (g0, g1, g2, g3, g4)
    x_blk = x_ref[...]
    s = lax.dot_general(
        x_blk, w0_ref[...], (((1,), (1,)), ((), ())), preferred_element_type=_f32
    )
    s = s + w0b_ref[...]
    hist = h_ref[...]
    wr = wr_ref[...]
    wrb = wrb_ref[...]
    denom = jnp.ones((BLK,), _f32)
    mask = jnp.zeros((BLK,), _i32)
    for r in range(R):
        gr = g_refs[r][...][0]
        s = s + lax.dot_general(
            gr, wr[r], (((1,), (1,)), ((), ())), preferred_element_type=_f32
        )
        d_r = hist[0, r, :]
        c_r = hist[0, R + r, :]
        s = s + d_r[:, None] * wrb[r][None, :]
        denom = denom + d_r
        mask = mask + ((d_r + c_r) == 0.0).astype(_i32)
    xo_ref[...] = jnp.maximum(s / denom[:, None], 0.0)
    m_ref[...] = mask[:, None]


def _tc_combine(x, g, hist3, wr_l, w0_l, wrb_l, w0b_l):
    g_spec = lambda r: pl.BlockSpec((1, BLK, D), lambda b, i, r=r: (b, r * NB + i, 0))
    return pl.pallas_call(
        _tc_body,
        grid=(B, NB),
        in_specs=[
            pl.BlockSpec((BLK, D), lambda b, i: (b * NB + i, 0)),
            g_spec(0), g_spec(1), g_spec(2), g_spec(3), g_spec(4),
            pl.BlockSpec((1, 2 * R, BLK), lambda b, i: (b, 0, i)),
            pl.BlockSpec((R, D, D), lambda b, i: (0, 0, 0)),
            pl.BlockSpec((D, D), lambda b, i: (0, 0)),
            pl.BlockSpec((R, D), lambda b, i: (0, 0)),
            pl.BlockSpec((1, D), lambda b, i: (0, 0)),
        ],
        out_specs=[
            pl.BlockSpec((BLK, D), lambda b, i: (b * NB + i, 0)),
            pl.BlockSpec((BLK, 1), lambda b, i: (b * NB + i, 0)),
        ],
        out_shape=[
            jax.ShapeDtypeStruct((B * NPAD, D), _f32),
            jax.ShapeDtypeStruct((B * NPAD, 1), _i32),
        ],
    )(x, g, g, g, g, g, hist3, wr_l, w0_l, wrb_l, w0b_l)


def kernel(nodes, edge_index, section, W0_w, W0_b, Wr_w, Wr_b):
    del section
    xpad = jnp.pad(nodes, ((0, 0), (0, NPAD - N), (0, 0)))
    x0 = xpad.reshape(B * NPAD, D)
    rows = edge_index[:, :, 0, :].astype(_i32)
    cols = edge_index[:, :, 1, :].astype(_i32)
    roff = (jnp.arange(R, dtype=_i32) * NPAD)[None, :, None]
    boff = (jnp.arange(B, dtype=_i32) * NPAD)[:, None, None]

    def tiled(a, nch, padv):
        a = a.reshape(B, NS, -1)
        pad = jnp.broadcast_to(padv, (B, NS, nch * 128 - a.shape[2])).astype(_i32)
        return jnp.concatenate([a, pad], axis=2)

    si = jnp.concatenate(
        [
            tiled((rows[:, :3] + roff[:, :3]).reshape(B, 3 * E), CH1, DUMP),
            tiled((rows[:, 3:] + roff[:, :2]).reshape(B, 2 * E), CH2, DUMP),
        ],
        axis=2,
    ).reshape(B * NS * CH, 128)
    gi = jnp.concatenate(
        [
            tiled((cols[:, :3] + boff).reshape(B, 3 * E), CH1, boff),
            tiled((cols[:, 3:] + boff).reshape(B, 2 * E), CH2, boff),
        ],
        axis=2,
    ).reshape(B * NS * CH, 128)
    hi = jnp.concatenate(
        [
            (rows + roff).reshape(B, R * E),
            (R * NPAD + cols + roff).reshape(B, R * E),
            jnp.full((B, NS * HEPT - 2 * R * E), HACC, _i32),
        ],
        axis=1,
    ).reshape(B * NS * HCH, 128)

    hist_raw = _sc_hist()(hi).reshape(B, HROWS)
    hist3 = hist_raw[:, : 2 * R * NPAD].reshape(B, 2 * R, NPAD)

    x = x0
    mask_flat = None
    es = jnp.concatenate([gi, si], axis=0)
    for l in range(L):
        g = _sc_g()(x, es)
        x, mask_flat = _tc_combine(
            x, g, hist3, Wr_w[:, l], W0_w[l], Wr_b[:, l], W0_b[l][None, :]
        )
    xout = x.reshape(B, NPAD, D)[:, :N]
    masks = mask_flat.reshape(B, NPAD)[:, :N]
    return (xout, masks)


# speedup vs baseline: 7.9323x; 1.0242x over previous
import functools

import jax
import jax.numpy as jnp
from jax import lax
from jax.experimental import pallas as pl
from jax.experimental.pallas import tpu as pltpu
from jax.experimental.pallas import tpu_sc as plsc

B, N, D = 4, 2500, 128
R, E, L = 5, 16000, 2
NPAD = 2560
NC, NS = 2, 16
BPC = B // NC
GROWS = R * NPAD
ACC_ROWS = 3 * NPAD + 128
DUMP = 3 * NPAD
ZPT = ACC_ROWS // NS
CH1, CH2 = 24, 16
CH = CH1 + CH2
HACC = 2 * R * NPAD
H_PT = 1664
HROWS = NS * H_PT
HEPT = 10240
HCH = HEPT // 128

_f32 = jnp.float32
_i32 = jnp.int32

@functools.lru_cache(maxsize=None)
def _mesh():
    return plsc.VectorSubcoreMesh(
        core_axis_name="c", subcore_axis_name="s", num_cores=NC, num_subcores=NS
    )


def _fill_zero(ref, nrows):
    @pl.loop(0, nrows)
    def _(i):
        for j in range(ref.shape[1] // 16):
            ref.at[pl.ds(i, 1), pl.ds(16 * j, 16)][...] = jnp.zeros((1, 16), _f32)


def _sc_g_body(x_hbm, es_hbm, g_hbm, gidx_v, sidx_v, buf0_v, buf1_v, buf2_v,
               acc_sh, sem0, sem1, sem2):
    c = lax.axis_index("c")
    s = lax.axis_index("s")
    zero_v = buf2_v
    bufs = ((buf0_v, sem0), (buf1_v, sem1), (buf2_v, sem2))
    DEPTH = len(bufs)

    def gfire(j, buf, sem):
        pltpu.async_copy(x_hbm.at[gidx_v.at[j]], buf, sem)

    def gwait(j, buf, sem):
        pltpu.make_async_copy(x_hbm.at[gidx_v.at[j]], buf, sem).wait()

    def scat(j, buf):
        pltpu.sync_copy(buf, acc_sh.at[sidx_v.at[j]], add=True)

    def run_chunks(lo, n):
        for j in range(min(DEPTH, n)):
            gfire(lo + j, *bufs[j % DEPTH])
        for j in range(n):
            buf, sem = bufs[j % DEPTH]
            gwait(lo + j, buf, sem)
            scat(lo + j, buf)
            if j + DEPTH < n:
                gfire(lo + j + DEPTH, buf, sem)
    def zero_acc(nrows_pt):
        base = pl.multiple_of(s * nrows_pt, 8)
        full, tail = nrows_pt // 128, nrows_pt % 128
        for j in range(full):
            pltpu.sync_copy(zero_v, acc_sh.at[pl.ds(base + 128 * j, 128)])
        if tail:
            pltpu.sync_copy(
                zero_v.at[pl.ds(0, tail)], acc_sh.at[pl.ds(base + 128 * full, tail)]
            )

    def writeout(b, nrows_pt, out_off):
        base = pl.multiple_of(s * nrows_pt, 8)
        full, tail = nrows_pt // 128, nrows_pt % 128
        for j in range(full):
            pltpu.sync_copy(
                acc_sh.at[pl.ds(base + 128 * j, 128)],
                g_hbm.at[b, pl.ds(out_off + base + 128 * j, 128)],
            )
        if tail:
            pltpu.sync_copy(
                acc_sh.at[pl.ds(base + 128 * full, tail)],
                g_hbm.at[b, pl.ds(out_off + base + 128 * full, tail)],
            )

    for k in range(BPC):
        b = c * BPC + k
        ebase = pl.multiple_of((b * NS + s) * CH, CH)
        pltpu.sync_copy(es_hbm.at[pl.ds(ebase, CH)], gidx_v)
        pltpu.sync_copy(es_hbm.at[pl.ds(B * NS * CH + ebase, CH)], sidx_v)
        _fill_zero(zero_v, 128)
        zero_acc(ZPT)
        plsc.subcore_barrier()

        run_chunks(0, CH1)
        plsc.subcore_barrier()
        writeout(b, 3 * NPAD // NS, 0)
        plsc.subcore_barrier()
        _fill_zero(zero_v, 128)
        zero_acc(2 * NPAD // NS)
        plsc.subcore_barrier()

        run_chunks(CH1, CH2)
        plsc.subcore_barrier()
        writeout(b, 2 * NPAD // NS, 3 * NPAD)
        plsc.subcore_barrier()


@functools.lru_cache(maxsize=None)
def _sc_g():
    return pl.kernel(
        _sc_g_body,
        out_type=jax.ShapeDtypeStruct((B, GROWS, D), _f32),
        mesh=_mesh(),
        scratch_types=[
            pltpu.VMEM((CH, 128), _i32),
            pltpu.VMEM((CH, 128), _i32),
            pltpu.VMEM((128, D), _f32),
            pltpu.VMEM((128, D), _f32),
            pltpu.VMEM((128, D), _f32),
            pltpu.VMEM_SHARED((ACC_ROWS, D), _f32),
            pltpu.SemaphoreType.DMA,
            pltpu.SemaphoreType.DMA,
            pltpu.SemaphoreType.DMA,
        ],
    )


def _sc_hist_body(hi_hbm, h_hbm, hidx_v, ones_v, zero_v, hacc_sh):
    c = lax.axis_index("c")
    s = lax.axis_index("s")

    @pl.loop(0, H_PT // 16)
    def _(i):
        zero_v.at[pl.ds(16 * i, 16)][...] = jnp.zeros((16,), _f32)

    @pl.loop(0, 8)
    def _(i):
        ones_v.at[pl.ds(16 * i, 16)][...] = jnp.ones((16,), _f32)

    for k in range(BPC):
        b = c * BPC + k
        base = pl.multiple_of(s * H_PT, H_PT)
        pltpu.sync_copy(zero_v, hacc_sh.at[pl.ds(base, H_PT)])
        plsc.subcore_barrier()
        hbase = pl.multiple_of((b * NS + s) * HCH, HCH)
        pltpu.sync_copy(hi_hbm.at[pl.ds(hbase, HCH)], hidx_v)

        @pl.loop(0, HCH)
        def _(j):
            pltpu.sync_copy(ones_v, hacc_sh.at[hidx_v.at[j]], add=True)

        plsc.subcore_barrier()
        hoff = pl.multiple_of(b * HROWS + base, 8)
        pltpu.sync_copy(hacc_sh.at[pl.ds(base, H_PT)], h_hbm.at[pl.ds(hoff, H_PT)])
        plsc.subcore_barrier()


@functools.lru_cache(maxsize=None)
def _sc_hist():
    return pl.kernel(
        _sc_hist_body,
        out_type=jax.ShapeDtypeStruct((B * HROWS,), _f32),
        mesh=_mesh(),
        scratch_types=[
            pltpu.VMEM((HCH, 128), _i32),
            pltpu.VMEM((128,), _f32),
            pltpu.VMEM((H_PT,), _f32),
            pltpu.VMEM_SHARED((HROWS,), _f32),
        ],
    )

BLK = 512
NB = NPAD // BLK


def _tc_body(x_ref, g0, g1, g2, g3, g4, h_ref, wr_ref, w0_ref, wrb_ref, w0b_ref,
             xo_ref, m_ref):
    g_refs = (g0, g1, g2, g3, g4)
    x_blk = x_ref[...]
    s = lax.dot_general(
        x_blk, w0_ref[...], (((1,), (1,)), ((), ())), preferred_element_type=_f32
    )
    s = s + w0b_ref[...]
    hist = h_ref[...]
    wr = wr_ref[...]
    wrb = wrb_ref[...]
    denom = jnp.ones((BLK,), _f32)
    mask = jnp.zeros((BLK,), _i32)
    for r in range(R):
        gr = g_refs[r][...][0]
        s = s + lax.dot_general(
            gr, wr[r], (((1,), (1,)), ((), ())), preferred_element_type=_f32
        )
        d_r = hist[0, r, :]
        c_r = hist[0, R + r, :]
        s = s + d_r[:, None] * wrb[r][None, :]
        denom = denom + d_r
        mask = mask + ((d_r + c_r) == 0.0).astype(_i32)
    xo_ref[...] = jnp.maximum(s / denom[:, None], 0.0)
    m_ref[...] = mask[:, None]


def _tc_combine(x, g, hist3, wr_l, w0_l, wrb_l, w0b_l):
    g_spec = lambda r: pl.BlockSpec((1, BLK, D), lambda b, i, r=r: (b, r * NB + i, 0))
    return pl.pallas_call(
        _tc_body,
        grid=(B, NB),
        in_specs=[
            pl.BlockSpec((BLK, D), lambda b, i: (b * NB + i, 0)),
            g_spec(0), g_spec(1), g_spec(2), g_spec(3), g_spec(4),
            pl.BlockSpec((1, 2 * R, BLK), lambda b, i: (b, 0, i)),
            pl.BlockSpec((R, D, D), lambda b, i: (0, 0, 0)),
            pl.BlockSpec((D, D), lambda b, i: (0, 0)),
            pl.BlockSpec((R, D), lambda b, i: (0, 0)),
            pl.BlockSpec((1, D), lambda b, i: (0, 0)),
        ],
        out_specs=[
            pl.BlockSpec((BLK, D), lambda b, i: (b * NB + i, 0)),
            pl.BlockSpec((BLK, 1), lambda b, i: (b * NB + i, 0)),
        ],
        out_shape=[
            jax.ShapeDtypeStruct((B * NPAD, D), _f32),
            jax.ShapeDtypeStruct((B * NPAD, 1), _i32),
        ],
    )(x, g, g, g, g, g, hist3, wr_l, w0_l, wrb_l, w0b_l)


def kernel(nodes, edge_index, section, W0_w, W0_b, Wr_w, Wr_b):
    del section
    xpad = jnp.pad(nodes, ((0, 0), (0, NPAD - N), (0, 0)))
    x0 = xpad.reshape(B * NPAD, D)
    rows = edge_index[:, :, 0, :].astype(_i32)
    cols = edge_index[:, :, 1, :].astype(_i32)
    roff = (jnp.arange(R, dtype=_i32) * NPAD)[None, :, None]
    boff = (jnp.arange(B, dtype=_i32) * NPAD)[:, None, None]

    def tiled(a, nch, padv):
        a = a.reshape(B, NS, -1)
        pad = jnp.broadcast_to(padv, (B, NS, nch * 128 - a.shape[2])).astype(_i32)
        return jnp.concatenate([a, pad], axis=2)

    si = jnp.concatenate(
        [
            tiled((rows[:, :3] + roff[:, :3]).reshape(B, 3 * E), CH1, DUMP),
            tiled((rows[:, 3:] + roff[:, :2]).reshape(B, 2 * E), CH2, DUMP),
        ],
        axis=2,
    ).reshape(B * NS * CH, 128)
    gi = jnp.concatenate(
        [
            tiled((cols[:, :3] + boff).reshape(B, 3 * E), CH1, boff),
            tiled((cols[:, 3:] + boff).reshape(B, 2 * E), CH2, boff),
        ],
        axis=2,
    ).reshape(B * NS * CH, 128)
    hi = jnp.concatenate(
        [
            (rows + roff).reshape(B, R * E),
            (R * NPAD + cols + roff).reshape(B, R * E),
            jnp.full((B, NS * HEPT - 2 * R * E), HACC, _i32),
        ],
        axis=1,
    ).reshape(B * NS * HCH, 128)

    hist_raw = _sc_hist()(hi).reshape(B, HROWS)
    hist3 = hist_raw[:, : 2 * R * NPAD].reshape(B, 2 * R, NPAD)

    x = x0
    mask_flat = None
    es = jnp.concatenate([gi, si], axis=0)
    for l in range(L):
        g = _sc_g()(x, es)
        x, mask_flat = _tc_combine(
            x, g, hist3, Wr_w[:, l], W0_w[l], Wr_b[:, l], W0_b[l][None, :]
        )
    xout = x.reshape(B, NPAD, D)[:, :N]
    masks = mask_flat.reshape(B, NPAD)[:, :N]
    return (xout, masks)
